# Initial kernel scaffold; baseline (speedup 1.0000x reference)
import jax
import jax.numpy as jnp
from jax.experimental import pallas as pl


def kernel(nodes, adj, weights, num_nodes, B, W1, b1, W2, b2):
    raise NotImplementedError("write your pallas kernel here")


# speedup vs baseline: 1.7060x; 1.7060x over previous
import functools

import numpy as np
import jax
import jax.numpy as jnp
from jax.experimental import pallas as pl
from jax.experimental.pallas import tpu as pltpu


_ROTS = ([13, 15, 26, 6], [17, 29, 16, 24])


def _np_threefry2x32(k0, k1, x0, x1):
    ks0 = np.uint32(k0)
    ks1 = np.uint32(k1)
    ks2 = np.uint32(ks0 ^ ks1 ^ np.uint32(0x1BD11BDA))
    ks = [ks0, ks1, ks2]
    x0 = (x0.astype(np.uint32) + ks0).astype(np.uint32)
    x1 = (x1.astype(np.uint32) + ks1).astype(np.uint32)
    for i in range(5):
        for r in _ROTS[i % 2]:
            x0 = (x0 + x1).astype(np.uint32)
            x1 = ((x1 << np.uint32(r)) | (x1 >> np.uint32(32 - r))).astype(np.uint32)
            x1 = (x1 ^ x0).astype(np.uint32)
        x0 = (x0 + ks[(i + 1) % 3]).astype(np.uint32)
        x1 = (x1 + ks[(i + 2) % 3] + np.uint32(i + 1)).astype(np.uint32)
    return x0, x1


def _as_i32(v):
    v = int(v) & 0xFFFFFFFF
    return v - 0x100000000 if v >= 0x80000000 else v


def _key_schedule(k0, k1):
    ks0 = np.uint32(k0)
    ks1 = np.uint32(k1)
    ks2 = np.uint32(ks0 ^ ks1 ^ np.uint32(0x1BD11BDA))
    ks = [ks0, ks1, ks2]
    init = (_as_i32(ks0), _as_i32(ks1))
    folds = tuple(
        (_as_i32(ks[(i + 1) % 3]), _as_i32(np.uint32(ks[(i + 2) % 3] + np.uint32(i + 1))))
        for i in range(5)
    )
    return init, folds


_B1, _B2 = _np_threefry2x32(0, 42, np.array([0, 0], np.uint32), np.array([0, 1], np.uint32))
_K1 = (int(_B1[0]), int(_B2[0]))
_K2 = (int(_B1[1]), int(_B2[1]))
_SCHED_E = _key_schedule(*_K1)
_SCHED_G = _key_schedule(*_K2)

_MIN_E = np.float32(1e-6)
_SCALE_E = np.float32(np.float32(1.0 - 1e-6) - np.float32(1e-6))
_MIN_G = np.float32(1e-10)
_SCALE_G = np.float32(np.float32(1.0) - np.float32(1e-10))
_A_HI = np.float32(np.float32(0.9999) / np.float32(1e-4))
_A_LO = np.float32(np.float32(1e-4) / np.float32(0.9999))


def _rotl(x, r):
    return jax.lax.shift_left(x, jnp.int32(r)) | jax.lax.shift_right_logical(
        x, jnp.int32(32 - r)
    )


def _tf_bits(sched, x1):
    (i0, i1), folds = sched
    x0 = jnp.int32(i0)
    x1 = x1 + jnp.int32(i1)
    for i, (a, b) in enumerate(folds):
        for r in _ROTS[i % 2]:
            x0 = x0 + x1
            x1 = _rotl(x1, r)
            x1 = x1 ^ x0
        x0 = x0 + jnp.int32(a)
        x1 = x1 + jnp.int32(b)
    return x0 ^ x1


def _to_unit(bits):
    fb = jax.lax.shift_right_logical(bits, jnp.int32(9)) | jnp.int32(0x3F800000)
    return jax.lax.bitcast_convert_type(fb, jnp.float32) - jnp.float32(1.0)


def _probs_kernel(nn_ref, nodes_ref, w1l_ref, w1r_ref, b1_ref, w2_ref, b2_ref,
                  prow_ref, pcol_ref):
    nn = nn_ref[pl.program_id(0)]
    nodes = nodes_ref[0]
    left = nodes_ref[0, pl.ds(nn, 1), :]
    hl = jnp.dot(left, w1l_ref[...], preferred_element_type=jnp.float32)
    h = jnp.dot(nodes, w1r_ref[...], preferred_element_type=jnp.float32)
    h = h + hl + b1_ref[...]
    h = jnp.where(h >= 0, h, jnp.float32(0.01) * h)
    b2 = b2_ref[0, 0]
    col_logit = jnp.dot(h, w2_ref[...], preferred_element_type=jnp.float32) + b2
    row_logit = jax.lax.dot_general(
        w2_ref[...], h, (((0,), (1,)), ((), ())),
        preferred_element_type=jnp.float32) + b2
    pcol_ref[0] = jnp.clip(jax.nn.sigmoid(col_logit), 1e-4, 0.9999)
    prow_ref[0] = jnp.clip(jax.nn.sigmoid(row_logit), 1e-4, 0.9999)


def _sample_kernel(nn_ref, w_ref, prow_ref, pcol_ref, adj_ref, wout_ref, *, n, ti, nb):
    b = pl.program_id(0)
    i = pl.program_id(1)
    nn = nn_ref[b]
    w_in = w_ref[0]
    prow = prow_ref[0]
    pcol = pcol_ref[0]

    rows = jax.lax.broadcasted_iota(jnp.int32, (ti, n), 0) + i * ti
    cols = jax.lax.broadcasted_iota(jnp.int32, (ti, n), 1)

    p = jnp.where((rows == nn) & (cols < nn), prow, w_in)
    p = jnp.where((cols == nn) & (rows < nn), pcol, p)
    wout_ref[0] = p

    lidx = (b * n + rows) * n + cols

    e = _to_unit(_tf_bits(_SCHED_E, lidx)) * _SCALE_E + _MIN_E
    e = jnp.maximum(e, _MIN_E)
    u0 = _to_unit(_tf_bits(_SCHED_G, lidx)) * _SCALE_G + _MIN_G
    u0 = jnp.maximum(u0, _MIN_G)
    u1 = _to_unit(_tf_bits(_SCHED_G, lidx + jnp.int32(nb * n * n))) * _SCALE_G + _MIN_G
    u1 = jnp.maximum(u1, _MIN_G)

    odds = (e * p) / ((jnp.float32(1.0) - e) * (jnp.float32(1.0) - p))
    odds = jnp.clip(odds, _A_LO, _A_HI)
    ratio = jnp.log(u0) / jnp.log(u1)
    adj_ref[0] = (odds * ratio > jnp.float32(1.0)).astype(jnp.float32)


def kernel(nodes, adj, weights, num_nodes, B, W1, b1, W2, b2):
    del adj, B
    bs, n, f = nodes.shape
    ti = 128
    nb = bs
    nn32 = num_nodes.astype(jnp.int32)

    prow, pcol = pl.pallas_call(
        _probs_kernel,
        grid=(bs,),
        in_specs=[
            pl.BlockSpec(memory_space=pltpu.SMEM),
            pl.BlockSpec((1, n, f), lambda b: (b, 0, 0)),
            pl.BlockSpec((f, f), lambda b: (0, 0)),
            pl.BlockSpec((f, f), lambda b: (0, 0)),
            pl.BlockSpec((1, f), lambda b: (0, 0)),
            pl.BlockSpec((f, 1), lambda b: (0, 0)),
            pl.BlockSpec((1, 1), lambda b: (0, 0)),
        ],
        out_specs=[
            pl.BlockSpec((1, 1, n), lambda b: (b, 0, 0)),
            pl.BlockSpec((1, n, 1), lambda b: (b, 0, 0)),
        ],
        out_shape=[
            jax.ShapeDtypeStruct((bs, 1, n), jnp.float32),
            jax.ShapeDtypeStruct((bs, n, 1), jnp.float32),
        ],
    )(nn32, nodes, W1[:f], W1[f:], b1.reshape(1, f), W2, b2.reshape(1, 1))

    adj_out, w_out = pl.pallas_call(
        functools.partial(_sample_kernel, n=n, ti=ti, nb=nb),
        grid=(bs, n // ti),
        in_specs=[
            pl.BlockSpec(memory_space=pltpu.SMEM),
            pl.BlockSpec((1, ti, n), lambda b, i: (b, i, 0)),
            pl.BlockSpec((1, 1, n), lambda b, i: (b, 0, 0)),
            pl.BlockSpec((1, ti, 1), lambda b, i: (b, i, 0)),
        ],
        out_specs=[
            pl.BlockSpec((1, ti, n), lambda b, i: (b, i, 0)),
            pl.BlockSpec((1, ti, n), lambda b, i: (b, i, 0)),
        ],
        out_shape=[
            jax.ShapeDtypeStruct((bs, n, n), jnp.float32),
            jax.ShapeDtypeStruct((bs, n, n), jnp.float32),
        ],
    )(nn32, weights, prow, pcol)

    return adj_out, w_out


# speedup vs baseline: 1.7060x; 1.0000x over previous
import functools

import numpy as np
import jax
import jax.numpy as jnp
from jax.experimental import pallas as pl
from jax.experimental.pallas import tpu as pltpu


_ROTS = ([13, 15, 26, 6], [17, 29, 16, 24])


def _np_threefry2x32(k0, k1, x0, x1):
    ks0 = np.uint32(k0)
    ks1 = np.uint32(k1)
    ks2 = np.uint32(ks0 ^ ks1 ^ np.uint32(0x1BD11BDA))
    ks = [ks0, ks1, ks2]
    x0 = (x0.astype(np.uint32) + ks0).astype(np.uint32)
    x1 = (x1.astype(np.uint32) + ks1).astype(np.uint32)
    for i in range(5):
        for r in _ROTS[i % 2]:
            x0 = (x0 + x1).astype(np.uint32)
            x1 = ((x1 << np.uint32(r)) | (x1 >> np.uint32(32 - r))).astype(np.uint32)
            x1 = (x1 ^ x0).astype(np.uint32)
        x0 = (x0 + ks[(i + 1) % 3]).astype(np.uint32)
        x1 = (x1 + ks[(i + 2) % 3] + np.uint32(i + 1)).astype(np.uint32)
    return x0, x1


def _as_i32(v):
    v = int(v) & 0xFFFFFFFF
    return v - 0x100000000 if v >= 0x80000000 else v


def _key_schedule(k0, k1):
    ks0 = np.uint32(k0)
    ks1 = np.uint32(k1)
    ks2 = np.uint32(ks0 ^ ks1 ^ np.uint32(0x1BD11BDA))
    ks = [ks0, ks1, ks2]
    init = (_as_i32(ks0), _as_i32(ks1))
    folds = tuple(
        (_as_i32(ks[(i + 1) % 3]), _as_i32(np.uint32(ks[(i + 2) % 3] + np.uint32(i + 1))))
        for i in range(5)
    )
    return init, folds


_B1, _B2 = _np_threefry2x32(0, 42, np.array([0, 0], np.uint32), np.array([0, 1], np.uint32))
_K1 = (int(_B1[0]), int(_B2[0]))
_K2 = (int(_B1[1]), int(_B2[1]))
_SCHED_E = _key_schedule(*_K1)
_SCHED_G = _key_schedule(*_K2)

_MIN_E = np.float32(1e-6)
_SCALE_E = np.float32(np.float32(1.0 - 1e-6) - np.float32(1e-6))
_MIN_G = np.float32(1e-10)
_SCALE_G = np.float32(np.float32(1.0) - np.float32(1e-10))
_A_HI = np.float32(np.float32(0.9999) / np.float32(1e-4))
_A_LO = np.float32(np.float32(1e-4) / np.float32(0.9999))


def _rotl(x, r):
    return jax.lax.shift_left(x, jnp.int32(r)) | jax.lax.shift_right_logical(
        x, jnp.int32(32 - r)
    )


def _tf_bits(sched, x1):
    (i0, i1), folds = sched
    x0 = jnp.int32(i0)
    x1 = x1 + jnp.int32(i1)
    for i, (a, b) in enumerate(folds):
        for r in _ROTS[i % 2]:
            x0 = x0 + x1
            x1 = _rotl(x1, r)
            x1 = x1 ^ x0
        x0 = x0 + jnp.int32(a)
        x1 = x1 + jnp.int32(b)
    return x0 ^ x1


def _to_unit(bits):
    fb = jax.lax.shift_right_logical(bits, jnp.int32(9)) | jnp.int32(0x3F800000)
    return jax.lax.bitcast_convert_type(fb, jnp.float32) - jnp.float32(1.0)


def _probs_kernel(nn_ref, nodes_ref, w1l_ref, w1r_ref, b1_ref, w2_ref, b2_ref,
                  prow_ref, pcol_ref):
    nn = nn_ref[pl.program_id(0)]
    nodes = nodes_ref[0]
    left = nodes_ref[0, pl.ds(nn, 1), :]
    hl = jnp.dot(left, w1l_ref[...], preferred_element_type=jnp.float32)
    h = jnp.dot(nodes, w1r_ref[...], preferred_element_type=jnp.float32)
    h = h + hl + b1_ref[...]
    h = jnp.where(h >= 0, h, jnp.float32(0.01) * h)
    b2 = b2_ref[0, 0]
    col_logit = jnp.dot(h, w2_ref[...], preferred_element_type=jnp.float32) + b2
    row_logit = jax.lax.dot_general(
        w2_ref[...], h, (((0,), (1,)), ((), ())),
        preferred_element_type=jnp.float32) + b2
    pcol_ref[0] = jnp.clip(jax.nn.sigmoid(col_logit), 1e-4, 0.9999)
    prow_ref[0] = jnp.clip(jax.nn.sigmoid(row_logit), 1e-4, 0.9999)


def _sample_kernel(nn_ref, w_ref, prow_ref, pcol_ref, adj_ref, wout_ref, *, n, ti, nb):
    b = pl.program_id(0)
    i = pl.program_id(1)
    nn = nn_ref[b]
    w_in = w_ref[0]
    prow = prow_ref[0]
    pcol = pcol_ref[0]

    rows = jax.lax.broadcasted_iota(jnp.int32, (ti, n), 0) + i * ti
    cols = jax.lax.broadcasted_iota(jnp.int32, (ti, n), 1)

    p = jnp.where((rows == nn) & (cols < nn), prow, w_in)
    p = jnp.where((cols == nn) & (rows < nn), pcol, p)
    wout_ref[0] = p

    lidx = (b * n + rows) * n + cols

    e = _to_unit(_tf_bits(_SCHED_E, lidx)) * _SCALE_E + _MIN_E
    e = jnp.maximum(e, _MIN_E)
    u0 = _to_unit(_tf_bits(_SCHED_G, lidx)) * _SCALE_G + _MIN_G
    u0 = jnp.maximum(u0, _MIN_G)
    u1 = _to_unit(_tf_bits(_SCHED_G, lidx + jnp.int32(nb * n * n))) * _SCALE_G + _MIN_G
    u1 = jnp.maximum(u1, _MIN_G)

    odds = (e * p) / ((jnp.float32(1.0) - e) * (jnp.float32(1.0) - p))
    odds = jnp.clip(odds, _A_LO, _A_HI)
    ratio = jnp.log(u0) / jnp.log(u1)
    adj_ref[0] = (odds * ratio > jnp.float32(1.0)).astype(jnp.float32)


def kernel(nodes, adj, weights, num_nodes, B, W1, b1, W2, b2):
    del adj, B
    bs, n, f = nodes.shape
    ti = 128
    nb = bs
    nn32 = num_nodes.astype(jnp.int32)

    prow, pcol = pl.pallas_call(
        _probs_kernel,
        grid=(bs,),
        in_specs=[
            pl.BlockSpec(memory_space=pltpu.SMEM),
            pl.BlockSpec((1, n, f), lambda b: (b, 0, 0)),
            pl.BlockSpec((f, f), lambda b: (0, 0)),
            pl.BlockSpec((f, f), lambda b: (0, 0)),
            pl.BlockSpec((1, f), lambda b: (0, 0)),
            pl.BlockSpec((f, 1), lambda b: (0, 0)),
            pl.BlockSpec((1, 1), lambda b: (0, 0)),
        ],
        out_specs=[
            pl.BlockSpec((1, 1, n), lambda b: (b, 0, 0)),
            pl.BlockSpec((1, n, 1), lambda b: (b, 0, 0)),
        ],
        out_shape=[
            jax.ShapeDtypeStruct((bs, 1, n), jnp.float32),
            jax.ShapeDtypeStruct((bs, n, 1), jnp.float32),
        ],
    )(nn32, nodes, W1[:f], W1[f:], b1.reshape(1, f), W2, b2.reshape(1, 1))

    adj_out, w_out = pl.pallas_call(
        functools.partial(_sample_kernel, n=n, ti=ti, nb=nb),
        grid=(bs, n // ti),
        in_specs=[
            pl.BlockSpec(memory_space=pltpu.SMEM),
            pl.BlockSpec((1, ti, n), lambda b, i: (b, i, 0)),
            pl.BlockSpec((1, 1, n), lambda b, i: (b, 0, 0)),
            pl.BlockSpec((1, ti, 1), lambda b, i: (b, i, 0)),
        ],
        out_specs=[
            pl.BlockSpec((1, ti, n), lambda b, i: (b, i, 0)),
            pl.BlockSpec((1, ti, n), lambda b, i: (b, i, 0)),
        ],
        out_shape=[
            jax.ShapeDtypeStruct((bs, n, n), jnp.float32),
            jax.ShapeDtypeStruct((bs, n, n), jnp.float32),
        ],
        compiler_params=pltpu.CompilerParams(
            dimension_semantics=("parallel", "parallel")),
    )(nn32, weights, prow, pcol)

    return adj_out, w_out


# speedup vs baseline: 2.7947x; 1.6381x over previous
import functools

import numpy as np
import jax
import jax.numpy as jnp
from jax.experimental import pallas as pl
from jax.experimental.pallas import tpu as pltpu


_ROTS = ([13, 15, 26, 6], [17, 29, 16, 24])


def _np_threefry2x32(k0, k1, x0, x1):
    ks0 = np.uint32(k0)
    ks1 = np.uint32(k1)
    ks2 = np.uint32(ks0 ^ ks1 ^ np.uint32(0x1BD11BDA))
    ks = [ks0, ks1, ks2]
    x0 = (x0.astype(np.uint32) + ks0).astype(np.uint32)
    x1 = (x1.astype(np.uint32) + ks1).astype(np.uint32)
    for i in range(5):
        for r in _ROTS[i % 2]:
            x0 = (x0 + x1).astype(np.uint32)
            x1 = ((x1 << np.uint32(r)) | (x1 >> np.uint32(32 - r))).astype(np.uint32)
            x1 = (x1 ^ x0).astype(np.uint32)
        x0 = (x0 + ks[(i + 1) % 3]).astype(np.uint32)
        x1 = (x1 + ks[(i + 2) % 3] + np.uint32(i + 1)).astype(np.uint32)
    return x0, x1


def _as_i32(v):
    v = int(v) & 0xFFFFFFFF
    return v - 0x100000000 if v >= 0x80000000 else v


def _key_schedule(k0, k1):
    ks0 = np.uint32(k0)
    ks1 = np.uint32(k1)
    ks2 = np.uint32(ks0 ^ ks1 ^ np.uint32(0x1BD11BDA))
    ks = [ks0, ks1, ks2]
    init = (_as_i32(ks0), _as_i32(ks1))
    folds = tuple(
        (_as_i32(ks[(i + 1) % 3]), _as_i32(np.uint32(ks[(i + 2) % 3] + np.uint32(i + 1))))
        for i in range(5)
    )
    return init, folds


_B1, _B2 = _np_threefry2x32(0, 42, np.array([0, 0], np.uint32), np.array([0, 1], np.uint32))
_K1 = (int(_B1[0]), int(_B2[0]))
_K2 = (int(_B1[1]), int(_B2[1]))
_SCHED_E = _key_schedule(*_K1)
_SCHED_G = _key_schedule(*_K2)

_MIN_E = np.float32(1e-6)
_SCALE_E = np.float32(np.float32(1.0 - 1e-6) - np.float32(1e-6))
_MIN_G = np.float32(1e-10)
_SCALE_G = np.float32(np.float32(1.0) - np.float32(1e-10))
_A_HI = np.float32(np.float32(0.9999) / np.float32(1e-4))
_A_LO = np.float32(np.float32(1e-4) / np.float32(0.9999))


def _rotl(x, r):
    return jax.lax.shift_left(x, jnp.int32(r)) | jax.lax.shift_right_logical(
        x, jnp.int32(32 - r)
    )


def _tf_bits(sched, x1):
    (i0, i1), folds = sched
    x0 = jnp.int32(i0)
    x1 = x1 + jnp.int32(i1)
    for i, (a, b) in enumerate(folds):
        for r in _ROTS[i % 2]:
            x0 = x0 + x1
            x1 = _rotl(x1, r)
            x1 = x1 ^ x0
        x0 = x0 + jnp.int32(a)
        x1 = x1 + jnp.int32(b)
    return x0 ^ x1


def _to_unit(bits):
    fb = jax.lax.shift_right_logical(bits, jnp.int32(9)) | jnp.int32(0x3F800000)
    return jax.lax.bitcast_convert_type(fb, jnp.float32) - jnp.float32(1.0)


def _probs_kernel(nn_ref, nodes_ref, w1l_ref, w1r_ref, b1_ref, w2_ref, b2_ref,
                  prow_ref, pcol_ref):
    nn = nn_ref[pl.program_id(0)]
    nodes = nodes_ref[0]
    left = nodes_ref[0, pl.ds(nn, 1), :]
    hl = jnp.dot(left, w1l_ref[...], preferred_element_type=jnp.float32)
    h = jnp.dot(nodes, w1r_ref[...], preferred_element_type=jnp.float32)
    h = h + hl + b1_ref[...]
    h = jnp.where(h >= 0, h, jnp.float32(0.01) * h)
    b2 = b2_ref[0, 0]
    col_logit = jnp.dot(h, w2_ref[...], preferred_element_type=jnp.float32) + b2
    row_logit = jax.lax.dot_general(
        w2_ref[...], h, (((0,), (1,)), ((), ())),
        preferred_element_type=jnp.float32) + b2
    pcol_ref[0] = jnp.clip(jax.nn.sigmoid(col_logit), 1e-4, 0.9999)
    prow_ref[0] = jnp.clip(jax.nn.sigmoid(row_logit), 1e-4, 0.9999)


def _sample_kernel(nn_ref, w_ref, prow_ref, pcol_ref, adj_ref, wout_ref,
                   *, n, ti, nb, cr, cw):
    b = pl.program_id(0)
    i = pl.program_id(1)
    nn = nn_ref[b]
    ncc = n // cw

    def body(ci, carry):
        r0 = (ci // ncc) * cr
        c0 = (ci % ncc) * cw
        rows = jax.lax.broadcasted_iota(jnp.int32, (cr, cw), 0) + (i * ti + r0)
        cols = jax.lax.broadcasted_iota(jnp.int32, (cr, cw), 1) + c0
        w_in = w_ref[0, pl.ds(r0, cr), pl.ds(c0, cw)]
        prow = prow_ref[0, :, pl.ds(c0, cw)]
        pcol = pcol_ref[0, pl.ds(r0, cr), :]

        p = jnp.where((rows == nn) & (cols < nn), prow, w_in)
        p = jnp.where((cols == nn) & (rows < nn), pcol, p)
        wout_ref[0, pl.ds(r0, cr), pl.ds(c0, cw)] = p

        lidx = (b * n + rows) * n + cols

        e = _to_unit(_tf_bits(_SCHED_E, lidx)) * _SCALE_E + _MIN_E
        u0 = _to_unit(_tf_bits(_SCHED_G, lidx)) * _SCALE_G + _MIN_G
        u1 = _to_unit(_tf_bits(_SCHED_G, lidx + jnp.int32(nb * n * n))) * _SCALE_G + _MIN_G

        odds = (e * p) / ((jnp.float32(1.0) - e) * (jnp.float32(1.0) - p))
        odds = jnp.clip(odds, _A_LO, _A_HI)
        ratio = jnp.log(u0) / jnp.log(u1)
        adj_ref[0, pl.ds(r0, cr), pl.ds(c0, cw)] = (
            odds * ratio > jnp.float32(1.0)).astype(jnp.float32)
        return carry

    jax.lax.fori_loop(0, (ti // cr) * ncc, body, 0, unroll=False)


def kernel(nodes, adj, weights, num_nodes, B, W1, b1, W2, b2):
    del adj, B
    bs, n, f = nodes.shape
    ti = 128
    nb = bs
    nn32 = num_nodes.astype(jnp.int32)

    prow, pcol = pl.pallas_call(
        _probs_kernel,
        grid=(bs,),
        in_specs=[
            pl.BlockSpec(memory_space=pltpu.SMEM),
            pl.BlockSpec((1, n, f), lambda b: (b, 0, 0)),
            pl.BlockSpec((f, f), lambda b: (0, 0)),
            pl.BlockSpec((f, f), lambda b: (0, 0)),
            pl.BlockSpec((1, f), lambda b: (0, 0)),
            pl.BlockSpec((f, 1), lambda b: (0, 0)),
            pl.BlockSpec((1, 1), lambda b: (0, 0)),
        ],
        out_specs=[
            pl.BlockSpec((1, 1, n), lambda b: (b, 0, 0)),
            pl.BlockSpec((1, n, 1), lambda b: (b, 0, 0)),
        ],
        out_shape=[
            jax.ShapeDtypeStruct((bs, 1, n), jnp.float32),
            jax.ShapeDtypeStruct((bs, n, 1), jnp.float32),
        ],
    )(nn32, nodes, W1[:f], W1[f:], b1.reshape(1, f), W2, b2.reshape(1, 1))

    adj_out, w_out = pl.pallas_call(
        functools.partial(_sample_kernel, n=n, ti=ti, nb=nb, cr=8, cw=2048),
        grid=(bs, n // ti),
        in_specs=[
            pl.BlockSpec(memory_space=pltpu.SMEM),
            pl.BlockSpec((1, ti, n), lambda b, i: (b, i, 0)),
            pl.BlockSpec((1, 1, n), lambda b, i: (b, 0, 0)),
            pl.BlockSpec((1, ti, 1), lambda b, i: (b, i, 0)),
        ],
        out_specs=[
            pl.BlockSpec((1, ti, n), lambda b, i: (b, i, 0)),
            pl.BlockSpec((1, ti, n), lambda b, i: (b, i, 0)),
        ],
        out_shape=[
            jax.ShapeDtypeStruct((bs, n, n), jnp.float32),
            jax.ShapeDtypeStruct((bs, n, n), jnp.float32),
        ],
        compiler_params=pltpu.CompilerParams(
            dimension_semantics=("parallel", "parallel")),
    )(nn32, weights, prow, pcol)

    return adj_out, w_out


# speedup vs baseline: 2.7980x; 1.0012x over previous
import functools

import numpy as np
import jax
import jax.numpy as jnp
from jax.experimental import pallas as pl
from jax.experimental.pallas import tpu as pltpu


_ROTS = ([13, 15, 26, 6], [17, 29, 16, 24])


def _np_threefry2x32(k0, k1, x0, x1):
    ks0 = np.uint32(k0)
    ks1 = np.uint32(k1)
    ks2 = np.uint32(ks0 ^ ks1 ^ np.uint32(0x1BD11BDA))
    ks = [ks0, ks1, ks2]
    x0 = (x0.astype(np.uint32) + ks0).astype(np.uint32)
    x1 = (x1.astype(np.uint32) + ks1).astype(np.uint32)
    for i in range(5):
        for r in _ROTS[i % 2]:
            x0 = (x0 + x1).astype(np.uint32)
            x1 = ((x1 << np.uint32(r)) | (x1 >> np.uint32(32 - r))).astype(np.uint32)
            x1 = (x1 ^ x0).astype(np.uint32)
        x0 = (x0 + ks[(i + 1) % 3]).astype(np.uint32)
        x1 = (x1 + ks[(i + 2) % 3] + np.uint32(i + 1)).astype(np.uint32)
    return x0, x1


def _as_i32(v):
    v = int(v) & 0xFFFFFFFF
    return v - 0x100000000 if v >= 0x80000000 else v


def _key_schedule(k0, k1):
    ks0 = np.uint32(k0)
    ks1 = np.uint32(k1)
    ks2 = np.uint32(ks0 ^ ks1 ^ np.uint32(0x1BD11BDA))
    ks = [ks0, ks1, ks2]
    init = (_as_i32(ks0), _as_i32(ks1))
    folds = tuple(
        (_as_i32(ks[(i + 1) % 3]), _as_i32(np.uint32(ks[(i + 2) % 3] + np.uint32(i + 1))))
        for i in range(5)
    )
    return init, folds


_B1, _B2 = _np_threefry2x32(0, 42, np.array([0, 0], np.uint32), np.array([0, 1], np.uint32))
_K1 = (int(_B1[0]), int(_B2[0]))
_K2 = (int(_B1[1]), int(_B2[1]))
_SCHED_E = _key_schedule(*_K1)
_SCHED_G = _key_schedule(*_K2)

_MIN_E = np.float32(1e-6)
_SCALE_E = np.float32(np.float32(1.0 - 1e-6) - np.float32(1e-6))
_MIN_G = np.float32(1e-10)
_SCALE_G = np.float32(np.float32(1.0) - np.float32(1e-10))
_A_HI = np.float32(np.float32(0.9999) / np.float32(1e-4))
_A_LO = np.float32(np.float32(1e-4) / np.float32(0.9999))


def _rotl(x, r):
    return jax.lax.shift_left(x, jnp.int32(r)) | jax.lax.shift_right_logical(
        x, jnp.int32(32 - r)
    )


def _tf_bits(sched, x1):
    (i0, i1), folds = sched
    x0 = jnp.int32(i0)
    x1 = x1 + jnp.int32(i1)
    for i, (a, b) in enumerate(folds):
        for r in _ROTS[i % 2]:
            x0 = x0 + x1
            x1 = _rotl(x1, r)
            x1 = x1 ^ x0
        x0 = x0 + jnp.int32(a)
        x1 = x1 + jnp.int32(b)
    return x0 ^ x1


def _to_unit(bits):
    fb = jax.lax.shift_right_logical(bits, jnp.int32(9)) | jnp.int32(0x3F800000)
    return jax.lax.bitcast_convert_type(fb, jnp.float32) - jnp.float32(1.0)


def _probs_kernel(nn_ref, nodes_ref, w1l_ref, w1r_ref, b1_ref, w2_ref, b2_ref,
                  prow_ref, pcol_ref):
    nn = nn_ref[pl.program_id(0)]
    nodes = nodes_ref[0]
    left = nodes_ref[0, pl.ds(nn, 1), :]
    hl = jnp.dot(left, w1l_ref[...], preferred_element_type=jnp.float32)
    h = jnp.dot(nodes, w1r_ref[...], preferred_element_type=jnp.float32)
    h = h + hl + b1_ref[...]
    h = jnp.where(h >= 0, h, jnp.float32(0.01) * h)
    b2 = b2_ref[0, 0]
    col_logit = jnp.dot(h, w2_ref[...], preferred_element_type=jnp.float32) + b2
    row_logit = jax.lax.dot_general(
        w2_ref[...], h, (((0,), (1,)), ((), ())),
        preferred_element_type=jnp.float32) + b2
    pcol_ref[0] = jnp.clip(jax.nn.sigmoid(col_logit), 1e-4, 0.9999)
    prow_ref[0] = jnp.clip(jax.nn.sigmoid(row_logit), 1e-4, 0.9999)


def _sample_kernel(nn_ref, w_ref, prow_ref, pcol_ref, adj_ref, wout_ref,
                   *, n, ti, nb, cr, cw):
    b = pl.program_id(0)
    i = pl.program_id(1)
    nn = nn_ref[b]
    ncc = n // cw

    def body(ci, carry):
        r0 = (ci // ncc) * cr
        c0 = (ci % ncc) * cw
        rows = jax.lax.broadcasted_iota(jnp.int32, (cr, cw), 0) + (i * ti + r0)
        cols = jax.lax.broadcasted_iota(jnp.int32, (cr, cw), 1) + c0
        w_in = w_ref[0, pl.ds(r0, cr), pl.ds(c0, cw)]
        prow = prow_ref[0, :, pl.ds(c0, cw)]
        pcol = pcol_ref[0, pl.ds(r0, cr), :]

        p = jnp.where((rows == nn) & (cols < nn), prow, w_in)
        p = jnp.where((cols == nn) & (rows < nn), pcol, p)
        wout_ref[0, pl.ds(r0, cr), pl.ds(c0, cw)] = p

        lidx = (b * n + rows) * n + cols

        e = _to_unit(_tf_bits(_SCHED_E, lidx)) * _SCALE_E + _MIN_E
        u0 = _to_unit(_tf_bits(_SCHED_G, lidx)) * _SCALE_G + _MIN_G
        u1 = _to_unit(_tf_bits(_SCHED_G, lidx + jnp.int32(nb * n * n))) * _SCALE_G + _MIN_G

        odds = (e * p) / ((jnp.float32(1.0) - e) * (jnp.float32(1.0) - p))
        odds = jnp.clip(odds, _A_LO, _A_HI)
        ratio = jnp.log(u0) / jnp.log(u1)
        adj_ref[0, pl.ds(r0, cr), pl.ds(c0, cw)] = (
            odds * ratio > jnp.float32(1.0)).astype(jnp.float32)
        return carry

    jax.lax.fori_loop(0, (ti // cr) * ncc, body, 0, unroll=False)


def kernel(nodes, adj, weights, num_nodes, B, W1, b1, W2, b2):
    del adj, B
    bs, n, f = nodes.shape
    ti = 256
    nb = bs
    nn32 = num_nodes.astype(jnp.int32)

    prow, pcol = pl.pallas_call(
        _probs_kernel,
        grid=(bs,),
        in_specs=[
            pl.BlockSpec(memory_space=pltpu.SMEM),
            pl.BlockSpec((1, n, f), lambda b: (b, 0, 0)),
            pl.BlockSpec((f, f), lambda b: (0, 0)),
            pl.BlockSpec((f, f), lambda b: (0, 0)),
            pl.BlockSpec((1, f), lambda b: (0, 0)),
            pl.BlockSpec((f, 1), lambda b: (0, 0)),
            pl.BlockSpec((1, 1), lambda b: (0, 0)),
        ],
        out_specs=[
            pl.BlockSpec((1, 1, n), lambda b: (b, 0, 0)),
            pl.BlockSpec((1, n, 1), lambda b: (b, 0, 0)),
        ],
        out_shape=[
            jax.ShapeDtypeStruct((bs, 1, n), jnp.float32),
            jax.ShapeDtypeStruct((bs, n, 1), jnp.float32),
        ],
    )(nn32, nodes, W1[:f], W1[f:], b1.reshape(1, f), W2, b2.reshape(1, 1))

    adj_out, w_out = pl.pallas_call(
        functools.partial(_sample_kernel, n=n, ti=ti, nb=nb, cr=8, cw=2048),
        grid=(bs, n // ti),
        in_specs=[
            pl.BlockSpec(memory_space=pltpu.SMEM),
            pl.BlockSpec((1, ti, n), lambda b, i: (b, i, 0)),
            pl.BlockSpec((1, 1, n), lambda b, i: (b, 0, 0)),
            pl.BlockSpec((1, ti, 1), lambda b, i: (b, i, 0)),
        ],
        out_specs=[
            pl.BlockSpec((1, ti, n), lambda b, i: (b, i, 0)),
            pl.BlockSpec((1, ti, n), lambda b, i: (b, i, 0)),
        ],
        out_shape=[
            jax.ShapeDtypeStruct((bs, n, n), jnp.float32),
            jax.ShapeDtypeStruct((bs, n, n), jnp.float32),
        ],
        compiler_params=pltpu.CompilerParams(
            dimension_semantics=("parallel", "parallel")),
    )(nn32, weights, prow, pcol)

    return adj_out, w_out


# speedup vs baseline: 2.8456x; 1.0170x over previous
import functools

import numpy as np
import jax
import jax.numpy as jnp
from jax.experimental import pallas as pl
from jax.experimental.pallas import tpu as pltpu


_ROTS = ([13, 15, 26, 6], [17, 29, 16, 24])


def _np_threefry2x32(k0, k1, x0, x1):
    ks0 = np.uint32(k0)
    ks1 = np.uint32(k1)
    ks2 = np.uint32(ks0 ^ ks1 ^ np.uint32(0x1BD11BDA))
    ks = [ks0, ks1, ks2]
    x0 = (x0.astype(np.uint32) + ks0).astype(np.uint32)
    x1 = (x1.astype(np.uint32) + ks1).astype(np.uint32)
    for i in range(5):
        for r in _ROTS[i % 2]:
            x0 = (x0 + x1).astype(np.uint32)
            x1 = ((x1 << np.uint32(r)) | (x1 >> np.uint32(32 - r))).astype(np.uint32)
            x1 = (x1 ^ x0).astype(np.uint32)
        x0 = (x0 + ks[(i + 1) % 3]).astype(np.uint32)
        x1 = (x1 + ks[(i + 2) % 3] + np.uint32(i + 1)).astype(np.uint32)
    return x0, x1


def _as_i32(v):
    v = int(v) & 0xFFFFFFFF
    return v - 0x100000000 if v >= 0x80000000 else v


def _key_schedule(k0, k1):
    ks0 = np.uint32(k0)
    ks1 = np.uint32(k1)
    ks2 = np.uint32(ks0 ^ ks1 ^ np.uint32(0x1BD11BDA))
    ks = [ks0, ks1, ks2]
    init = (_as_i32(ks0), _as_i32(ks1))
    folds = tuple(
        (_as_i32(ks[(i + 1) % 3]), _as_i32(np.uint32(ks[(i + 2) % 3] + np.uint32(i + 1))))
        for i in range(5)
    )
    return init, folds


_B1, _B2 = _np_threefry2x32(0, 42, np.array([0, 0], np.uint32), np.array([0, 1], np.uint32))
_K1 = (int(_B1[0]), int(_B2[0]))
_K2 = (int(_B1[1]), int(_B2[1]))
_SCHED_E = _key_schedule(*_K1)
_SCHED_G = _key_schedule(*_K2)

_MIN_E = np.float32(1e-6)
_SCALE_E = np.float32(np.float32(1.0 - 1e-6) - np.float32(1e-6))
_MIN_G = np.float32(1e-10)
_SCALE_G = np.float32(np.float32(1.0) - np.float32(1e-10))
_A_HI = np.float32(np.float32(0.9999) / np.float32(1e-4))
_A_LO = np.float32(np.float32(1e-4) / np.float32(0.9999))


def _rotl(x, r):
    return jax.lax.shift_left(x, jnp.int32(r)) | jax.lax.shift_right_logical(
        x, jnp.int32(32 - r)
    )


def _tf_bits(sched, x1):
    (i0, i1), folds = sched
    x0 = jnp.int32(i0)
    x1 = x1 + jnp.int32(i1)
    for i, (a, b) in enumerate(folds):
        for r in _ROTS[i % 2]:
            x0 = x0 + x1
            x1 = _rotl(x1, r)
            x1 = x1 ^ x0
        x0 = x0 + jnp.int32(a)
        x1 = x1 + jnp.int32(b)
    return x0 ^ x1


def _to_unit(bits):
    fb = jax.lax.shift_right_logical(bits, jnp.int32(9)) | jnp.int32(0x3F800000)
    return jax.lax.bitcast_convert_type(fb, jnp.float32) - jnp.float32(1.0)


def _probs_kernel(nn_ref, nodes_ref, w1l_ref, w1r_ref, b1_ref, w2_ref, b2_ref,
                  prow_ref, pcol_ref):
    nn = nn_ref[pl.program_id(0)]
    nodes = nodes_ref[0]
    left = nodes_ref[0, pl.ds(nn, 1), :]
    hl = jnp.dot(left, w1l_ref[...], preferred_element_type=jnp.float32)
    h = jnp.dot(nodes, w1r_ref[...], preferred_element_type=jnp.float32)
    h = h + hl + b1_ref[...]
    h = jnp.where(h >= 0, h, jnp.float32(0.01) * h)
    b2 = b2_ref[0, 0]
    col_logit = jnp.dot(h, w2_ref[...], preferred_element_type=jnp.float32) + b2
    row_logit = jax.lax.dot_general(
        w2_ref[...], h, (((0,), (1,)), ((), ())),
        preferred_element_type=jnp.float32) + b2
    pcol_ref[0] = jnp.clip(jax.nn.sigmoid(col_logit), 1e-4, 0.9999)
    prow_ref[0] = jnp.clip(jax.nn.sigmoid(row_logit), 1e-4, 0.9999)


def _sample_kernel(nn_ref, w_ref, prow_ref, pcol_ref, adj_ref, wout_ref,
                   *, n, ti, nb, cr, cw):
    b = pl.program_id(0)
    i = pl.program_id(1)
    nn = nn_ref[b]
    ncc = n // cw

    def body(ci, carry):
        r0 = (ci // ncc) * cr
        c0 = (ci % ncc) * cw
        rows = jax.lax.broadcasted_iota(jnp.int32, (cr, cw), 0) + (i * ti + r0)
        cols = jax.lax.broadcasted_iota(jnp.int32, (cr, cw), 1) + c0
        w_in = w_ref[0, pl.ds(r0, cr), pl.ds(c0, cw)]
        prow = prow_ref[0, :, pl.ds(c0, cw)]
        pcol = pcol_ref[0, pl.ds(r0, cr), :]

        p = jnp.where((rows == nn) & (cols < nn), prow, w_in)
        p = jnp.where((cols == nn) & (rows < nn), pcol, p)
        wout_ref[0, pl.ds(r0, cr), pl.ds(c0, cw)] = p

        lidx = (b * n + rows) * n + cols

        e = _to_unit(_tf_bits(_SCHED_E, lidx)) * _SCALE_E + _MIN_E
        u0 = _to_unit(_tf_bits(_SCHED_G, lidx)) * _SCALE_G + _MIN_G
        u1 = _to_unit(_tf_bits(_SCHED_G, lidx + jnp.int32(nb * n * n))) * _SCALE_G + _MIN_G

        odds = (e * p) / ((jnp.float32(1.0) - e) * (jnp.float32(1.0) - p))
        odds = jnp.clip(odds, _A_LO, _A_HI)
        ratio = jnp.log(u0) / jnp.log(u1)
        adj_ref[0, pl.ds(r0, cr), pl.ds(c0, cw)] = (
            odds * ratio > jnp.float32(1.0)).astype(jnp.float32)
        return carry

    jax.lax.fori_loop(0, (ti // cr) * ncc, body, 0, unroll=8)


def kernel(nodes, adj, weights, num_nodes, B, W1, b1, W2, b2):
    del adj, B
    bs, n, f = nodes.shape
    ti = 256
    nb = bs
    nn32 = num_nodes.astype(jnp.int32)

    prow, pcol = pl.pallas_call(
        _probs_kernel,
        grid=(bs,),
        in_specs=[
            pl.BlockSpec(memory_space=pltpu.SMEM),
            pl.BlockSpec((1, n, f), lambda b: (b, 0, 0)),
            pl.BlockSpec((f, f), lambda b: (0, 0)),
            pl.BlockSpec((f, f), lambda b: (0, 0)),
            pl.BlockSpec((1, f), lambda b: (0, 0)),
            pl.BlockSpec((f, 1), lambda b: (0, 0)),
            pl.BlockSpec((1, 1), lambda b: (0, 0)),
        ],
        out_specs=[
            pl.BlockSpec((1, 1, n), lambda b: (b, 0, 0)),
            pl.BlockSpec((1, n, 1), lambda b: (b, 0, 0)),
        ],
        out_shape=[
            jax.ShapeDtypeStruct((bs, 1, n), jnp.float32),
            jax.ShapeDtypeStruct((bs, n, 1), jnp.float32),
        ],
    )(nn32, nodes, W1[:f], W1[f:], b1.reshape(1, f), W2, b2.reshape(1, 1))

    adj_out, w_out = pl.pallas_call(
        functools.partial(_sample_kernel, n=n, ti=ti, nb=nb, cr=8, cw=2048),
        grid=(bs, n // ti),
        in_specs=[
            pl.BlockSpec(memory_space=pltpu.SMEM),
            pl.BlockSpec((1, ti, n), lambda b, i: (b, i, 0)),
            pl.BlockSpec((1, 1, n), lambda b, i: (b, 0, 0)),
            pl.BlockSpec((1, ti, 1), lambda b, i: (b, i, 0)),
        ],
        out_specs=[
            pl.BlockSpec((1, ti, n), lambda b, i: (b, i, 0)),
            pl.BlockSpec((1, ti, n), lambda b, i: (b, i, 0)),
        ],
        out_shape=[
            jax.ShapeDtypeStruct((bs, n, n), jnp.float32),
            jax.ShapeDtypeStruct((bs, n, n), jnp.float32),
        ],
        compiler_params=pltpu.CompilerParams(
            dimension_semantics=("parallel", "parallel")),
    )(nn32, weights, prow, pcol)

    return adj_out, w_out


# speedup vs baseline: 2.8947x; 1.0172x over previous
import functools

import numpy as np
import jax
import jax.numpy as jnp
from jax.experimental import pallas as pl
from jax.experimental.pallas import tpu as pltpu


_ROTS = ([13, 15, 26, 6], [17, 29, 16, 24])


def _np_threefry2x32(k0, k1, x0, x1):
    ks0 = np.uint32(k0)
    ks1 = np.uint32(k1)
    ks2 = np.uint32(ks0 ^ ks1 ^ np.uint32(0x1BD11BDA))
    ks = [ks0, ks1, ks2]
    x0 = (x0.astype(np.uint32) + ks0).astype(np.uint32)
    x1 = (x1.astype(np.uint32) + ks1).astype(np.uint32)
    for i in range(5):
        for r in _ROTS[i % 2]:
            x0 = (x0 + x1).astype(np.uint32)
            x1 = ((x1 << np.uint32(r)) | (x1 >> np.uint32(32 - r))).astype(np.uint32)
            x1 = (x1 ^ x0).astype(np.uint32)
        x0 = (x0 + ks[(i + 1) % 3]).astype(np.uint32)
        x1 = (x1 + ks[(i + 2) % 3] + np.uint32(i + 1)).astype(np.uint32)
    return x0, x1


def _as_i32(v):
    v = int(v) & 0xFFFFFFFF
    return v - 0x100000000 if v >= 0x80000000 else v


def _key_schedule(k0, k1):
    ks0 = np.uint32(k0)
    ks1 = np.uint32(k1)
    ks2 = np.uint32(ks0 ^ ks1 ^ np.uint32(0x1BD11BDA))
    ks = [ks0, ks1, ks2]
    init = (_as_i32(ks0), _as_i32(ks1))
    folds = tuple(
        (_as_i32(ks[(i + 1) % 3]), _as_i32(np.uint32(ks[(i + 2) % 3] + np.uint32(i + 1))))
        for i in range(5)
    )
    return init, folds


_B1, _B2 = _np_threefry2x32(0, 42, np.array([0, 0], np.uint32), np.array([0, 1], np.uint32))
_K1 = (int(_B1[0]), int(_B2[0]))
_K2 = (int(_B1[1]), int(_B2[1]))
_SCHED_E = _key_schedule(*_K1)
_SCHED_G = _key_schedule(*_K2)

_MIN_E = np.float32(1e-6)
_SCALE_E = np.float32(np.float32(1.0 - 1e-6) - np.float32(1e-6))
_MIN_G = np.float32(1e-10)
_SCALE_G = np.float32(np.float32(1.0) - np.float32(1e-10))
assert _SCALE_G == np.float32(1.0)
_A_HI = np.float32(np.float32(0.9999) / np.float32(1e-4))
_A_LO = np.float32(np.float32(1e-4) / np.float32(0.9999))


def _rotl(x, r):
    return jax.lax.shift_left(x, jnp.int32(r)) | jax.lax.shift_right_logical(
        x, jnp.int32(32 - r)
    )


def _tf_bits(sched, x1):
    (i0, _), folds = sched
    x0 = jnp.int32(i0)
    for i, (a, b) in enumerate(folds):
        for r in _ROTS[i % 2]:
            x0 = x0 + x1
            x1 = _rotl(x1, r)
            x1 = x1 ^ x0
        x0 = x0 + jnp.int32(a)
        x1 = x1 + jnp.int32(b)
    return x0 ^ x1


def _to_unit(bits):
    fb = jax.lax.shift_right_logical(bits, jnp.int32(9)) | jnp.int32(0x3F800000)
    return jax.lax.bitcast_convert_type(fb, jnp.float32) - jnp.float32(1.0)


def _probs_kernel(nn_ref, nodes_ref, w1l_ref, w1r_ref, b1_ref, w2_ref, b2_ref,
                  prow_ref, pcol_ref):
    nn = nn_ref[pl.program_id(0)]
    nodes = nodes_ref[0]
    left = nodes_ref[0, pl.ds(nn, 1), :]
    hl = jnp.dot(left, w1l_ref[...], preferred_element_type=jnp.float32)
    h = jnp.dot(nodes, w1r_ref[...], preferred_element_type=jnp.float32)
    h = h + hl + b1_ref[...]
    h = jnp.where(h >= 0, h, jnp.float32(0.01) * h)
    b2 = b2_ref[0, 0]
    col_logit = jnp.dot(h, w2_ref[...], preferred_element_type=jnp.float32) + b2
    row_logit = jax.lax.dot_general(
        w2_ref[...], h, (((0,), (1,)), ((), ())),
        preferred_element_type=jnp.float32) + b2
    pcol_ref[0] = jnp.clip(jax.nn.sigmoid(col_logit), 1e-4, 0.9999)
    prow_ref[0] = jnp.clip(jax.nn.sigmoid(row_logit), 1e-4, 0.9999)


def _sample_kernel(nn_ref, w_ref, prow_ref, pcol_ref, adj_ref, wout_ref,
                   *, n, ti, nb, cr, cw):
    b = pl.program_id(0)
    i = pl.program_id(1)
    nn = nn_ref[b]
    ncc = n // cw
    i1e = _SCHED_E[0][1]
    i1g = _SCHED_G[0][1]

    riota = jax.lax.broadcasted_iota(jnp.int32, (cr, cw), 0)
    ciota = jax.lax.broadcasted_iota(jnp.int32, (cr, cw), 1)
    pat = riota * jnp.int32(n) + ciota

    def body(ci, carry):
        r0 = (ci // ncc) * cr
        c0 = (ci % ncc) * cw
        rowbase = i * ti + r0
        nnr = nn - rowbase
        base = (b * n + rowbase) * n + c0
        w_in = w_ref[0, pl.ds(r0, cr), pl.ds(c0, cw)]
        prow = prow_ref[0, :, pl.ds(c0, cw)]
        pcol = pcol_ref[0, pl.ds(r0, cr), :]

        p = jnp.where((riota == nnr) & (ciota < nn), prow, w_in)
        p = jnp.where((ciota == nn) & (riota < nnr), pcol, p)
        wout_ref[0, pl.ds(r0, cr), pl.ds(c0, cw)] = p

        e = _to_unit(_tf_bits(_SCHED_E, pat + (base + jnp.int32(i1e)))) * _SCALE_E + _MIN_E
        u0 = _to_unit(_tf_bits(_SCHED_G, pat + (base + jnp.int32(i1g)))) + _MIN_G
        u1 = _to_unit(_tf_bits(_SCHED_G, pat + (base + jnp.int32(i1g) + jnp.int32(nb * n * n)))) + _MIN_G

        odds = (e * p) / ((jnp.float32(1.0) - e) * (jnp.float32(1.0) - p))
        odds = jnp.clip(odds, _A_LO, _A_HI)
        ratio = jnp.log(u0) / jnp.log(u1)
        adj_ref[0, pl.ds(r0, cr), pl.ds(c0, cw)] = (
            odds * ratio > jnp.float32(1.0)).astype(jnp.float32)
        return carry

    jax.lax.fori_loop(0, (ti // cr) * ncc, body, 0, unroll=8)


def kernel(nodes, adj, weights, num_nodes, B, W1, b1, W2, b2):
    del adj, B
    bs, n, f = nodes.shape
    ti = 512
    nb = bs
    nn32 = num_nodes.astype(jnp.int32)

    prow, pcol = pl.pallas_call(
        _probs_kernel,
        grid=(bs,),
        in_specs=[
            pl.BlockSpec(memory_space=pltpu.SMEM),
            pl.BlockSpec((1, n, f), lambda b: (b, 0, 0)),
            pl.BlockSpec((f, f), lambda b: (0, 0)),
            pl.BlockSpec((f, f), lambda b: (0, 0)),
            pl.BlockSpec((1, f), lambda b: (0, 0)),
            pl.BlockSpec((f, 1), lambda b: (0, 0)),
            pl.BlockSpec((1, 1), lambda b: (0, 0)),
        ],
        out_specs=[
            pl.BlockSpec((1, 1, n), lambda b: (b, 0, 0)),
            pl.BlockSpec((1, n, 1), lambda b: (b, 0, 0)),
        ],
        out_shape=[
            jax.ShapeDtypeStruct((bs, 1, n), jnp.float32),
            jax.ShapeDtypeStruct((bs, n, 1), jnp.float32),
        ],
    )(nn32, nodes, W1[:f], W1[f:], b1.reshape(1, f), W2, b2.reshape(1, 1))

    adj_out, w_out = pl.pallas_call(
        functools.partial(_sample_kernel, n=n, ti=ti, nb=nb, cr=8, cw=2048),
        grid=(bs, n // ti),
        in_specs=[
            pl.BlockSpec(memory_space=pltpu.SMEM),
            pl.BlockSpec((1, ti, n), lambda b, i: (b, i, 0)),
            pl.BlockSpec((1, 1, n), lambda b, i: (b, 0, 0)),
            pl.BlockSpec((1, ti, 1), lambda b, i: (b, i, 0)),
        ],
        out_specs=[
            pl.BlockSpec((1, ti, n), lambda b, i: (b, i, 0)),
            pl.BlockSpec((1, ti, n), lambda b, i: (b, i, 0)),
        ],
        out_shape=[
            jax.ShapeDtypeStruct((bs, n, n), jnp.float32),
            jax.ShapeDtypeStruct((bs, n, n), jnp.float32),
        ],
        compiler_params=pltpu.CompilerParams(
            dimension_semantics=("arbitrary", "arbitrary")),
    )(nn32, weights, prow, pcol)

    return adj_out, w_out
